# baseline (speedup 1.0000x reference)
import functools

import jax
import jax.numpy as jnp
from jax import lax
from jax.experimental import pallas as pl
from jax.experimental.pallas import tpu as pltpu
from jax.experimental.pallas import tpu_sc as plsc

NC = 2
NS = 16
NW = NC * NS
CHUNK = 128
NBUF = 3
IR = 6


def _sc_aggregate(n_rows, d, m_chunks):
    stripe = -(-n_rows // NS)
    stripe += (-stripe) % 8
    tail = n_rows - stripe * (NS - 1)
    assert tail > 0 and tail % 8 == 0 and stripe % 8 == 0
    base_c = m_chunks // NW
    extra = m_chunks - base_c * NW

    mesh = plsc.VectorSubcoreMesh(core_axis_name="c", subcore_axis_name="s")

    @functools.partial(
        pl.kernel,
        out_type=jax.ShapeDtypeStruct((NC, n_rows, d), jnp.float32),
        mesh=mesh,
        scratch_types=[
            pltpu.VMEM((IR, 1, CHUNK), jnp.int32),
            pltpu.VMEM((IR, 1, CHUNK), jnp.int32),
            pltpu.VMEM((CHUNK, d), jnp.float32),
            pltpu.VMEM((CHUNK, d), jnp.float32),
            pltpu.VMEM((CHUNK, d), jnp.float32),
            pltpu.SemaphoreType.DMA,
            pltpu.SemaphoreType.DMA,
            pltpu.SemaphoreType.DMA,
            pltpu.SemaphoreType.DMA,
            pltpu.SemaphoreType.DMA,
            pltpu.SemaphoreType.DMA,
            pltpu.SemaphoreType.DMA,
            pltpu.SemaphoreType.DMA,
            pltpu.SemaphoreType.DMA,
            pltpu.SemaphoreType.DMA,
            pltpu.SemaphoreType.DMA,
            pltpu.SemaphoreType.DMA,
            pltpu.VMEM_SHARED((n_rows, d), jnp.float32),
        ],
    )
    def sc_kernel(x_hbm, src_hbm, dst_hbm, zeros_hbm, out_hbm,
                  src_r, dst_r, b0, b1, b2,
                  i0, i1, i2, i3, i4, i5, g0, g1, g2, c0, c1, c2, acc):
        c = lax.axis_index("c")
        s = lax.axis_index("s")
        wid = s * NC + c
        bufs = (b0, b1, b2)
        gsems = (g0, g1, g2)
        ssems = (c0, c1, c2)
        isems = (i0, i1, i2, i3, i4, i5)

        start = base_c * wid + jnp.minimum(wid, extra)
        n_real = base_c + jnp.where(wid < extra, 1, 0)
        t_main = base_c - (base_c % IR)
        ep_max = (base_c % IR) + 1
        row0 = s * stripe

        @pl.when(s < NS - 1)
        def _():
            pltpu.sync_copy(zeros_hbm, acc.at[pl.ds(row0, stripe)])

        @pl.when(s == NS - 1)
        def _():
            pltpu.sync_copy(zeros_hbm.at[pl.ds(0, tail)],
                            acc.at[pl.ds((NS - 1) * stripe, tail)])

        def grow(t):
            return jnp.minimum(start + t, m_chunks - 1)

        def idx_load(t, slot):
            g = grow(t)
            pltpu.make_async_copy(src_hbm.at[g], src_r.at[slot],
                                  isems[slot]).start()
            pltpu.make_async_copy(dst_hbm.at[g], dst_r.at[slot],
                                  isems[slot]).start()

        def idx_wait(slot):
            pltpu.make_async_copy(src_hbm.at[0], src_r.at[slot],
                                  isems[slot]).wait()
            pltpu.make_async_copy(dst_hbm.at[0], dst_r.at[slot],
                                  isems[slot]).wait()

        def gather(islot, ring):
            return pltpu.make_async_copy(
                x_hbm.at[src_r.at[islot, 0]], bufs[ring], gsems[ring])

        def scat_start(islot, ring):
            pltpu.async_copy(bufs[ring], acc.at[dst_r.at[islot, 0]],
                             ssems[ring], add=True)

        def scat_wait(ring):
            pltpu.make_async_copy(bufs[ring], acc.at[dst_r.at[0, 0]],
                                  ssems[ring]).wait()

        for t in range(IR):
            idx_load(t, t)
        for t in range(2):
            idx_wait(t)
        plsc.subcore_barrier()
        for t in range(2):
            gather(t, t).start()

        @pl.loop(0, t_main, step=IR)
        def _(t):
            for r in range(IR):
                tt = t + r
                ring = r % NBUF
                gather(r, ring).wait()

                @pl.when(tt < n_real)
                def _():
                    scat_start(r, ring)

                nring = (r + 2) % NBUF
                nslot = (r + 2) % IR
                fslot = (r + 5) % IR

                @pl.when(tt + 2 < n_real)
                def _():
                    @pl.when(tt >= 1)
                    def _():
                        scat_wait(nring)

                        @pl.when(tt + 5 < n_real)
                        def _():
                            idx_load(tt + 5, fslot)

                    idx_wait(nslot)
                    gather(nslot, nring).start()

        for r_e in range(ep_max):
            tt_e = t_main + r_e
            ring_e = tt_e % NBUF
            islot_e = tt_e % IR

            @pl.when(tt_e < n_real)
            def _():
                gather(islot_e, ring_e).wait()
                scat_start(islot_e, ring_e)

        for ring in range(NBUF):
            scat_wait(ring)

        plsc.subcore_barrier()
        @pl.when(s < NS - 1)
        def _():
            pltpu.sync_copy(acc.at[pl.ds(row0, stripe)],
                            out_hbm.at[c, pl.ds(row0, stripe)])

        @pl.when(s == NS - 1)
        def _():
            pltpu.sync_copy(acc.at[pl.ds((NS - 1) * stripe, tail)],
                            out_hbm.at[c, pl.ds((NS - 1) * stripe, tail)])

    return sc_kernel


def _combine_body(x_ref, p0_ref, p1_ref, w_ref, b_ref, o_ref):
    agg = p0_ref[0] + p1_ref[0]
    conv = lax.dot_general(
        agg, w_ref[...], (((1,), (0,)), ((), ())),
        precision=lax.Precision.HIGHEST,
        preferred_element_type=jnp.float32,
    )
    o_ref[...] = x_ref[...] + conv + b_ref[...]


@jax.jit
def kernel(x, edge_index, W, b):
    n, d = x.shape
    e = edge_index.shape[1]
    m_chunks = e // CHUNK

    src3 = edge_index[0].reshape(m_chunks, 1, CHUNK)
    dst3 = edge_index[1].reshape(m_chunks, 1, CHUNK)

    stripe = -(-n // NS)
    stripe += (-stripe) % 8
    zeros = jnp.zeros((stripe, d), jnp.float32)

    partials = _sc_aggregate(n, d, m_chunks)(x, src3, dst3, zeros)

    blk = 2000
    nb = n // blk
    b2 = b.reshape(1, d)
    row_spec = pl.BlockSpec((blk, d), lambda i: (i, 0))
    out = pl.pallas_call(
        _combine_body,
        grid=(nb,),
        in_specs=[
            row_spec,
            pl.BlockSpec((1, blk, d), lambda i: (0, i, 0)),
            pl.BlockSpec((1, blk, d), lambda i: (1, i, 0)),
            pl.BlockSpec((d, d), lambda i: (0, 0)),
            pl.BlockSpec((1, d), lambda i: (0, 0)),
        ],
        out_specs=row_spec,
        out_shape=jax.ShapeDtypeStruct((n, d), jnp.float32),
    )(x, partials, partials, W, b2)
    return out


# speedup vs baseline: 1.1047x; 1.1047x over previous
import functools

import jax
import jax.numpy as jnp
from jax import lax
from jax.experimental import pallas as pl
from jax.experimental.pallas import tpu as pltpu
from jax.experimental.pallas import tpu_sc as plsc

NC = 2
NS = 16
NW = NC * NS
CHUNK = 128
NBUF = 3
IR = 6


def _sc_aggregate(n_rows, d, m_chunks):
    stripe = -(-n_rows // NS)
    stripe += (-stripe) % 8
    tail = n_rows - stripe * (NS - 1)
    assert tail > 0 and tail % 8 == 0 and stripe % 8 == 0
    base_c = m_chunks // NW
    extra = m_chunks - base_c * NW

    mesh = plsc.VectorSubcoreMesh(core_axis_name="c", subcore_axis_name="s")

    @functools.partial(
        pl.kernel,
        out_type=jax.ShapeDtypeStruct((NC, n_rows, d), jnp.float32),
        mesh=mesh,
        scratch_types=[
            pltpu.VMEM((IR * CHUNK,), jnp.int32),
            pltpu.VMEM((IR, CHUNK), jnp.int32),
            pltpu.VMEM((CHUNK, d), jnp.float32),
            pltpu.VMEM((CHUNK, d), jnp.float32),
            pltpu.VMEM((CHUNK, d), jnp.float32),
            pltpu.SemaphoreType.DMA,
            pltpu.SemaphoreType.DMA,
            pltpu.SemaphoreType.DMA,
            pltpu.SemaphoreType.DMA,
            pltpu.SemaphoreType.DMA,
            pltpu.SemaphoreType.DMA,
            pltpu.SemaphoreType.DMA,
            pltpu.SemaphoreType.DMA,
            pltpu.SemaphoreType.DMA,
            pltpu.VMEM_SHARED((n_rows, d), jnp.float32),
        ],
    )
    def sc_kernel(x_hbm, ei_hbm, zeros_hbm, out_hbm,
                  src_r, dst_r, b0, b1, b2,
                  i0, i1, i2, i3, i4, i5, g0, g1, g2, acc):
        c = lax.axis_index("c")
        s = lax.axis_index("s")
        wid = s * NC + c
        bufs = (b0, b1, b2)
        gsems = (g0, g1, g2)
        isems = (i0, i1, i2, i3, i4, i5)

        start = base_c * wid + jnp.minimum(wid, extra)
        n_real = base_c + jnp.where(wid < extra, 1, 0)
        t_main = base_c - (base_c % IR)
        ep_max = (base_c % IR) + 1
        row0 = s * stripe

        @pl.when(s < NS - 1)
        def _():
            pltpu.sync_copy(zeros_hbm, acc.at[pl.ds(row0, stripe)])

        @pl.when(s == NS - 1)
        def _():
            pltpu.sync_copy(zeros_hbm.at[pl.ds(0, tail)],
                            acc.at[pl.ds((NS - 1) * stripe, tail)])

        def grow(t):
            return jnp.minimum(start + t, m_chunks - 1)

        def idx_load(t, slot):
            off = grow(t) * CHUNK
            pltpu.make_async_copy(ei_hbm.at[0, pl.ds(off, CHUNK)],
                                  src_r.at[pl.ds(slot * CHUNK, CHUNK)],
                                  isems[slot]).start()
            pltpu.make_async_copy(ei_hbm.at[1, pl.ds(off, CHUNK)],
                                  dst_r.at[slot], isems[slot]).start()

        def idx_wait(slot):
            pltpu.make_async_copy(ei_hbm.at[0, pl.ds(0, CHUNK)],
                                  src_r.at[pl.ds(slot * CHUNK, CHUNK)],
                                  isems[slot]).wait()
            pltpu.make_async_copy(ei_hbm.at[1, pl.ds(0, CHUNK)],
                                  dst_r.at[slot], isems[slot]).wait()

        def gather(islot, ring):
            return pltpu.make_async_copy(
                x_hbm.at[src_r.at[pl.ds(islot * CHUNK, CHUNK)]],
                bufs[ring], gsems[ring])

        for t in range(IR):
            idx_load(t, t)
        for t in range(NBUF):
            idx_wait(t)
        plsc.subcore_barrier()
        for t in range(NBUF):
            gather(t, t).start()

        @pl.loop(0, t_main, step=IR)
        def _(t):
            for r in range(IR):
                tt = t + r
                ring = r % NBUF
                gather(r, ring).wait()

                @pl.when(tt < n_real)
                def _():
                    pltpu.sync_copy(bufs[ring], acc.at[dst_r.at[r]],
                                    add=True)

                @pl.when(tt + IR < n_real)
                def _():
                    idx_load(tt + IR, r)

                nslot = (r + NBUF) % IR

                @pl.when(tt + NBUF < n_real)
                def _():
                    idx_wait(nslot)
                    gather(nslot, ring).start()

        for r_e in range(ep_max):
            tt_e = t_main + r_e
            ring_e = tt_e % NBUF
            islot_e = tt_e % IR

            @pl.when(tt_e < n_real)
            def _():
                gather(islot_e, ring_e).wait()
                pltpu.sync_copy(bufs[ring_e], acc.at[dst_r.at[islot_e]],
                                add=True)

        plsc.subcore_barrier()
        @pl.when(s < NS - 1)
        def _():
            pltpu.sync_copy(acc.at[pl.ds(row0, stripe)],
                            out_hbm.at[c, pl.ds(row0, stripe)])

        @pl.when(s == NS - 1)
        def _():
            pltpu.sync_copy(acc.at[pl.ds((NS - 1) * stripe, tail)],
                            out_hbm.at[c, pl.ds((NS - 1) * stripe, tail)])

    return sc_kernel


def _combine_body(x_ref, p0_ref, p1_ref, w_ref, b_ref, o_ref):
    agg = p0_ref[0] + p1_ref[0]
    conv = lax.dot_general(
        agg, w_ref[...], (((1,), (0,)), ((), ())),
        precision=lax.Precision.HIGHEST,
        preferred_element_type=jnp.float32,
    )
    o_ref[...] = x_ref[...] + conv + b_ref[...]


@jax.jit
def kernel(x, edge_index, W, b):
    n, d = x.shape
    e = edge_index.shape[1]
    m_chunks = e // CHUNK

    stripe = -(-n // NS)
    stripe += (-stripe) % 8
    zeros = jnp.zeros((stripe, d), jnp.float32)

    partials = _sc_aggregate(n, d, m_chunks)(x, edge_index, zeros)

    blk = 2000
    nb = n // blk
    b2 = b.reshape(1, d)
    row_spec = pl.BlockSpec((blk, d), lambda i: (i, 0))
    out = pl.pallas_call(
        _combine_body,
        grid=(nb,),
        in_specs=[
            row_spec,
            pl.BlockSpec((1, blk, d), lambda i: (0, i, 0)),
            pl.BlockSpec((1, blk, d), lambda i: (1, i, 0)),
            pl.BlockSpec((d, d), lambda i: (0, 0)),
            pl.BlockSpec((1, d), lambda i: (0, 0)),
        ],
        out_specs=row_spec,
        out_shape=jax.ShapeDtypeStruct((n, d), jnp.float32),
    )(x, partials, partials, W, b2)
    return out


# speedup vs baseline: 1.1444x; 1.0359x over previous
import functools

import jax
import jax.numpy as jnp
from jax import lax
from jax.experimental import pallas as pl
from jax.experimental.pallas import tpu as pltpu
from jax.experimental.pallas import tpu_sc as plsc

NC = 2
NS = 16
NW = NC * NS
CHUNK = 128
NBUF = 3
IR = 6


def _sc_aggregate(n_rows, d, m_chunks):
    stripe = -(-n_rows // NS)
    stripe += (-stripe) % 8
    tail = n_rows - stripe * (NS - 1)
    assert tail > 0 and tail % 8 == 0 and stripe % 8 == 0
    base_c = m_chunks // NW
    extra = m_chunks - base_c * NW

    mesh = plsc.VectorSubcoreMesh(core_axis_name="c", subcore_axis_name="s")

    @functools.partial(
        pl.kernel,
        out_type=jax.ShapeDtypeStruct((NC, n_rows, d), jnp.float32),
        mesh=mesh,
        scratch_types=[
            pltpu.VMEM((IR * CHUNK,), jnp.int32),
            pltpu.VMEM((IR, CHUNK), jnp.int32),
            pltpu.VMEM((CHUNK, d), jnp.float32),
            pltpu.VMEM((CHUNK, d), jnp.float32),
            pltpu.VMEM((CHUNK, d), jnp.float32),
            pltpu.SemaphoreType.DMA,
            pltpu.SemaphoreType.DMA,
            pltpu.SemaphoreType.DMA,
            pltpu.SemaphoreType.DMA,
            pltpu.SemaphoreType.DMA,
            pltpu.SemaphoreType.DMA,
            pltpu.SemaphoreType.DMA,
            pltpu.SemaphoreType.DMA,
            pltpu.SemaphoreType.DMA,
            pltpu.VMEM_SHARED((n_rows, d), jnp.float32),
        ],
    )
    def sc_kernel(x_hbm, ei_hbm, out_hbm,
                  src_r, dst_r, b0, b1, b2,
                  i0, i1, i2, i3, i4, i5, g0, g1, g2, acc):
        c = lax.axis_index("c")
        s = lax.axis_index("s")
        wid = s * NC + c
        bufs = (b0, b1, b2)
        gsems = (g0, g1, g2)
        isems = (i0, i1, i2, i3, i4, i5)

        start = base_c * wid + jnp.minimum(wid, extra)
        n_real = base_c + jnp.where(wid < extra, 1, 0)
        t_main = base_c - (base_c % IR)
        ep_max = (base_c % IR) + 1
        row0 = s * stripe

        z16 = jnp.zeros((16,), jnp.float32)

        @pl.loop(0, CHUNK)
        def _(i):
            for j in range(d // 16):
                b0[i, pl.ds(j * 16, 16)] = z16

        my_rows = jnp.where(s < NS - 1, stripe, tail)

        @pl.loop(0, my_rows, step=CHUNK)
        def _(k):
            nrows = jnp.minimum(my_rows - k, CHUNK)

            @pl.when(nrows >= CHUNK)
            def _():
                pltpu.sync_copy(b0, acc.at[pl.ds(row0 + k, CHUNK)])

            @pl.when(nrows < CHUNK)
            def _():
                if stripe % CHUNK:
                    @pl.when(s < NS - 1)
                    def _():
                        pltpu.sync_copy(b0.at[pl.ds(0, stripe % CHUNK)],
                                        acc.at[pl.ds(row0 + k,
                                                     stripe % CHUNK)])

                if tail % CHUNK:
                    @pl.when(s == NS - 1)
                    def _():
                        pltpu.sync_copy(b0.at[pl.ds(0, tail % CHUNK)],
                                        acc.at[pl.ds(row0 + k,
                                                     tail % CHUNK)])

        def grow(t):
            return jnp.minimum(start + t, m_chunks - 1)

        def idx_load(t, slot):
            off = grow(t) * CHUNK
            pltpu.make_async_copy(ei_hbm.at[0, pl.ds(off, CHUNK)],
                                  src_r.at[pl.ds(slot * CHUNK, CHUNK)],
                                  isems[slot]).start()
            pltpu.make_async_copy(ei_hbm.at[1, pl.ds(off, CHUNK)],
                                  dst_r.at[slot], isems[slot]).start()

        def idx_wait(slot):
            pltpu.make_async_copy(ei_hbm.at[0, pl.ds(0, CHUNK)],
                                  src_r.at[pl.ds(slot * CHUNK, CHUNK)],
                                  isems[slot]).wait()
            pltpu.make_async_copy(ei_hbm.at[1, pl.ds(0, CHUNK)],
                                  dst_r.at[slot], isems[slot]).wait()

        def gather(islot, ring):
            return pltpu.make_async_copy(
                x_hbm.at[src_r.at[pl.ds(islot * CHUNK, CHUNK)]],
                bufs[ring], gsems[ring])

        for t in range(IR):
            idx_load(t, t)
        for t in range(NBUF):
            idx_wait(t)
        plsc.subcore_barrier()
        for t in range(NBUF):
            gather(t, t).start()

        @pl.loop(0, t_main, step=IR)
        def _(t):
            for r in range(IR):
                tt = t + r
                ring = r % NBUF
                gather(r, ring).wait()

                @pl.when(tt < n_real)
                def _():
                    pltpu.sync_copy(bufs[ring], acc.at[dst_r.at[r]],
                                    add=True)

                @pl.when(tt + IR < n_real)
                def _():
                    idx_load(tt + IR, r)

                nslot = (r + NBUF) % IR

                @pl.when(tt + NBUF < n_real)
                def _():
                    idx_wait(nslot)
                    gather(nslot, ring).start()

        for r_e in range(ep_max):
            tt_e = t_main + r_e
            ring_e = tt_e % NBUF
            islot_e = tt_e % IR

            @pl.when(tt_e < n_real)
            def _():
                gather(islot_e, ring_e).wait()
                pltpu.sync_copy(bufs[ring_e], acc.at[dst_r.at[islot_e]],
                                add=True)

        plsc.subcore_barrier()
        @pl.when(s < NS - 1)
        def _():
            pltpu.sync_copy(acc.at[pl.ds(row0, stripe)],
                            out_hbm.at[c, pl.ds(row0, stripe)])

        @pl.when(s == NS - 1)
        def _():
            pltpu.sync_copy(acc.at[pl.ds((NS - 1) * stripe, tail)],
                            out_hbm.at[c, pl.ds((NS - 1) * stripe, tail)])

    return sc_kernel


def _combine_body(x_ref, p0_ref, p1_ref, w_ref, b_ref, o_ref):
    agg = p0_ref[0] + p1_ref[0]
    conv = lax.dot_general(
        agg, w_ref[...], (((1,), (0,)), ((), ())),
        precision=lax.Precision.HIGHEST,
        preferred_element_type=jnp.float32,
    )
    o_ref[...] = x_ref[...] + conv + b_ref[...]


@jax.jit
def kernel(x, edge_index, W, b):
    n, d = x.shape
    e = edge_index.shape[1]
    m_chunks = e // CHUNK

    partials = _sc_aggregate(n, d, m_chunks)(x, edge_index)

    blk = 2000
    nb = n // blk
    b2 = b.reshape(1, d)
    row_spec = pl.BlockSpec((blk, d), lambda i: (i, 0))
    out = pl.pallas_call(
        _combine_body,
        grid=(nb,),
        in_specs=[
            row_spec,
            pl.BlockSpec((1, blk, d), lambda i: (0, i, 0)),
            pl.BlockSpec((1, blk, d), lambda i: (1, i, 0)),
            pl.BlockSpec((d, d), lambda i: (0, 0)),
            pl.BlockSpec((1, d), lambda i: (0, 0)),
        ],
        out_specs=row_spec,
        out_shape=jax.ShapeDtypeStruct((n, d), jnp.float32),
    )(x, partials, partials, W, b2)
    return out


# speedup vs baseline: 1.1638x; 1.0170x over previous
import functools

import jax
import jax.numpy as jnp
from jax import lax
from jax.experimental import pallas as pl
from jax.experimental.pallas import tpu as pltpu
from jax.experimental.pallas import tpu_sc as plsc

NC = 2
NS = 16
NW = NC * NS
CHUNK = 128
NBUF = 3
IR = 6


def _sc_aggregate(n_rows, d, m_chunks):
    stripe = -(-n_rows // NS)
    stripe += (-stripe) % 8
    tail = n_rows - stripe * (NS - 1)
    assert tail > 0 and tail % 8 == 0 and stripe % 8 == 0
    base_c = m_chunks // NW
    extra = m_chunks - base_c * NW

    mesh = plsc.VectorSubcoreMesh(core_axis_name="c", subcore_axis_name="s")

    @functools.partial(
        pl.kernel,
        out_type=jax.ShapeDtypeStruct((NC, n_rows, d), jnp.float32),
        mesh=mesh,
        scratch_types=[
            pltpu.VMEM((IR * CHUNK,), jnp.int32),
            pltpu.VMEM((IR, CHUNK), jnp.int32),
            pltpu.VMEM((CHUNK, d), jnp.float32),
            pltpu.VMEM((CHUNK, d), jnp.float32),
            pltpu.VMEM((CHUNK, d), jnp.float32),
            pltpu.SemaphoreType.DMA,
            pltpu.SemaphoreType.DMA,
            pltpu.SemaphoreType.DMA,
            pltpu.SemaphoreType.DMA,
            pltpu.SemaphoreType.DMA,
            pltpu.SemaphoreType.DMA,
            pltpu.SemaphoreType.DMA,
            pltpu.SemaphoreType.DMA,
            pltpu.SemaphoreType.DMA,
            pltpu.VMEM_SHARED((n_rows, d), jnp.float32),
        ],
    )
    def sc_kernel(x_hbm, ei_hbm, out_hbm,
                  src_r, dst_r, b0, b1, b2,
                  i0, i1, i2, i3, i4, i5, g0, g1, g2, acc):
        c = lax.axis_index("c")
        s = lax.axis_index("s")
        wid = s * NC + c
        bufs = (b0, b1, b2)
        gsems = (g0, g1, g2)
        isems = (i0, i1, i2, i3, i4, i5)

        start = base_c * wid + jnp.minimum(wid, extra)
        n_real = base_c + jnp.where(wid < extra, 1, 0)
        t_main = base_c - (base_c % IR)
        ep_max = (base_c % IR) + 1
        row0 = s * stripe

        z16 = jnp.zeros((16,), jnp.float32)

        @pl.loop(0, CHUNK)
        def _(i):
            for j in range(d // 16):
                b0[i, pl.ds(j * 16, 16)] = z16

        my_rows = jnp.where(s < NS - 1, stripe, tail)

        @pl.loop(0, my_rows, step=CHUNK)
        def _(k):
            nrows = jnp.minimum(my_rows - k, CHUNK)

            @pl.when(nrows >= CHUNK)
            def _():
                pltpu.sync_copy(b0, acc.at[pl.ds(row0 + k, CHUNK)])

            @pl.when(nrows < CHUNK)
            def _():
                if stripe % CHUNK:
                    @pl.when(s < NS - 1)
                    def _():
                        pltpu.sync_copy(b0.at[pl.ds(0, stripe % CHUNK)],
                                        acc.at[pl.ds(row0 + k,
                                                     stripe % CHUNK)])

                if tail % CHUNK:
                    @pl.when(s == NS - 1)
                    def _():
                        pltpu.sync_copy(b0.at[pl.ds(0, tail % CHUNK)],
                                        acc.at[pl.ds(row0 + k,
                                                     tail % CHUNK)])

        def grow(t):
            return jnp.minimum(start + t, m_chunks - 1)

        def idx_load(t, slot):
            off = grow(t) * CHUNK
            pltpu.make_async_copy(ei_hbm.at[0, pl.ds(off, CHUNK)],
                                  src_r.at[pl.ds(slot * CHUNK, CHUNK)],
                                  isems[slot]).start()
            pltpu.make_async_copy(ei_hbm.at[1, pl.ds(off, CHUNK)],
                                  dst_r.at[slot], isems[slot]).start()

        def idx_wait(slot):
            pltpu.make_async_copy(ei_hbm.at[0, pl.ds(0, CHUNK)],
                                  src_r.at[pl.ds(slot * CHUNK, CHUNK)],
                                  isems[slot]).wait()
            pltpu.make_async_copy(ei_hbm.at[1, pl.ds(0, CHUNK)],
                                  dst_r.at[slot], isems[slot]).wait()

        def gather(islot, ring):
            return pltpu.make_async_copy(
                x_hbm.at[src_r.at[pl.ds(islot * CHUNK, CHUNK)]],
                bufs[ring], gsems[ring])

        for t in range(IR):
            idx_load(t, t)
        for t in range(NBUF):
            idx_wait(t)
        plsc.subcore_barrier()
        for t in range(NBUF):
            gather(t, t).start()

        @pl.loop(0, t_main, step=IR)
        def _(t):
            for r in range(IR):
                tt = t + r
                ring = r % NBUF
                gather(r, ring).wait()

                @pl.when(tt < n_real)
                def _():
                    pltpu.sync_copy(bufs[ring], acc.at[dst_r.at[r]],
                                    add=True)

                @pl.when(tt + IR < n_real)
                def _():
                    idx_load(tt + IR, r)

                nslot = (r + NBUF) % IR

                @pl.when(tt + NBUF < n_real)
                def _():
                    idx_wait(nslot)
                    gather(nslot, ring).start()

        for r_e in range(ep_max):
            tt_e = t_main + r_e
            ring_e = tt_e % NBUF
            islot_e = tt_e % IR

            @pl.when(tt_e < n_real)
            def _():
                gather(islot_e, ring_e).wait()
                pltpu.sync_copy(bufs[ring_e], acc.at[dst_r.at[islot_e]],
                                add=True)

        plsc.subcore_barrier()
        @pl.when(s < NS - 1)
        def _():
            pltpu.sync_copy(acc.at[pl.ds(row0, stripe)],
                            out_hbm.at[c, pl.ds(row0, stripe)])

        @pl.when(s == NS - 1)
        def _():
            pltpu.sync_copy(acc.at[pl.ds((NS - 1) * stripe, tail)],
                            out_hbm.at[c, pl.ds((NS - 1) * stripe, tail)])

    return sc_kernel


def _combine_body(x_ref, p0_ref, p1_ref, w_ref, b_ref, o_ref):
    agg = p0_ref[0] + p1_ref[0]
    conv = lax.dot_general(
        agg, w_ref[...], (((1,), (0,)), ((), ())),
        preferred_element_type=jnp.float32,
    )
    o_ref[...] = x_ref[...] + conv + b_ref[...]


@jax.jit
def kernel(x, edge_index, W, b):
    n, d = x.shape
    e = edge_index.shape[1]
    m_chunks = e // CHUNK

    partials = _sc_aggregate(n, d, m_chunks)(x, edge_index)

    blk = 2000
    nb = n // blk
    b2 = b.reshape(1, d)
    row_spec = pl.BlockSpec((blk, d), lambda i: (i, 0))
    out = pl.pallas_call(
        _combine_body,
        grid=(nb,),
        in_specs=[
            row_spec,
            pl.BlockSpec((1, blk, d), lambda i: (0, i, 0)),
            pl.BlockSpec((1, blk, d), lambda i: (1, i, 0)),
            pl.BlockSpec((d, d), lambda i: (0, 0)),
            pl.BlockSpec((1, d), lambda i: (0, 0)),
        ],
        out_specs=row_spec,
        out_shape=jax.ShapeDtypeStruct((n, d), jnp.float32),
    )(x, partials, partials, W, b2)
    return out
